# Initial kernel scaffold; baseline (speedup 1.0000x reference)
import jax
import jax.numpy as jnp
from jax.experimental import pallas as pl


def kernel(x, adj_row, adj_col, adj_val, W1, b1, W2, b2):
    raise NotImplementedError("write your pallas kernel here")


# speedup vs baseline: 3.9544x; 3.9544x over previous
import jax
import jax.numpy as jnp
from jax import lax
from jax.experimental import pallas as pl
from jax.experimental.pallas import tpu as pltpu
from jax.experimental.pallas import tpu_sc as plsc

_N = 10000
_E = 320000
_NC = 2
_NS = 16
_NW = _NC * _NS
_EPW = _E // _NW
_C = 80
_NCHUNK = _EPW // _C
_RPT = _N // _NS
_ZR = 125
_L = 16


def _make_spmm(feat):
    grp = feat // _L
    mesh = plsc.VectorSubcoreMesh(core_axis_name="c", subcore_axis_name="s")

    def body(y_hbm, row_hbm, col_hbm, val_hbm, out_hbm,
             colv, valv, rowv, rows, zbuf, acc, sem):
        cid = lax.axis_index("c")
        sid = lax.axis_index("s")
        wid = cid * _NS + sid

        @pl.loop(0, _ZR)
        def _(r):
            for k in range(grp):
                zbuf[r, pl.ds(k * _L, _L)] = jnp.zeros((_L,), jnp.float32)

        @pl.loop(0, _RPT // _ZR)
        def _(z):
            pltpu.sync_copy(zbuf, acc.at[pl.ds(sid * _RPT + z * _ZR, _ZR)])

        plsc.subcore_barrier()

        @pl.loop(0, _NCHUNK)
        def _(i):
            base = wid * _EPW + i * _C
            pltpu.sync_copy(col_hbm.at[pl.ds(base, _C)], colv)
            pltpu.sync_copy(val_hbm.at[pl.ds(base, _C)], valv)
            pltpu.sync_copy(row_hbm.at[pl.ds(base, _C)], rowv)
            pltpu.async_copy(y_hbm.at[colv], rows, sem).wait()

            @pl.loop(0, _C // _L)
            def _(j0):
                vals16 = valv[pl.ds(j0 * _L, _L)]
                for t in range(_L):
                    v = jnp.take(vals16, jnp.full((_L,), t, jnp.int32))
                    j = j0 * _L + t
                    for k in range(grp):
                        sl = pl.ds(k * _L, _L)
                        rows[j, sl] = rows[j, sl] * v

            pltpu.sync_copy(rows, acc.at[rowv], add=True)

        plsc.subcore_barrier()
        pltpu.sync_copy(acc.at[pl.ds(sid * _RPT, _RPT)],
                        out_hbm.at[cid, pl.ds(sid * _RPT, _RPT)])

    return pl.kernel(
        body,
        out_type=jax.ShapeDtypeStruct((_NC, _N, feat), jnp.float32),
        mesh=mesh,
        scratch_types=[
            pltpu.VMEM((_C,), jnp.int32),
            pltpu.VMEM((_C,), jnp.float32),
            pltpu.VMEM((_C,), jnp.int32),
            pltpu.VMEM((_C, feat), jnp.float32),
            pltpu.VMEM((_ZR, feat), jnp.float32),
            pltpu.VMEM_SHARED((_N, feat), jnp.float32),
            pltpu.SemaphoreType.DMA,
        ],
        compiler_params=pltpu.CompilerParams(use_tc_tiling_on_sc=False),
    )


_spmm128 = _make_spmm(128)
_spmm64 = _make_spmm(64)

_BM = 2000


def _mm1(x, w1):
    def body(x_ref, w_ref, o_ref):
        o_ref[...] = lax.dot_general(
            x_ref[...], w_ref[...], (((1,), (1,)), ((), ())),
            preferred_element_type=jnp.float32)

    return pl.pallas_call(
        body,
        grid=(_N // _BM,),
        in_specs=[pl.BlockSpec((_BM, 128), lambda i: (i, 0)),
                  pl.BlockSpec((128, 128), lambda i: (0, 0))],
        out_specs=pl.BlockSpec((_BM, 128), lambda i: (i, 0)),
        out_shape=jax.ShapeDtypeStruct((_N, 128), jnp.float32),
    )(x, w1)


def _mm2(zp, w2, b1r):
    def body(z_ref, w_ref, b_ref, o_ref):
        z = z_ref[0] + z_ref[1] + b_ref[...]
        o_ref[...] = lax.dot_general(
            z, w_ref[...], (((1,), (1,)), ((), ())),
            preferred_element_type=jnp.float32)

    return pl.pallas_call(
        body,
        grid=(_N // _BM,),
        in_specs=[pl.BlockSpec((2, _BM, 128), lambda i: (0, i, 0)),
                  pl.BlockSpec((64, 128), lambda i: (0, 0)),
                  pl.BlockSpec((1, 128), lambda i: (0, 0))],
        out_specs=pl.BlockSpec((_BM, 64), lambda i: (i, 0)),
        out_shape=jax.ShapeDtypeStruct((_N, 64), jnp.float32),
    )(zp, w2, b1r)


def _add2(op, b2r):
    def body(o_ref, b_ref, out_ref):
        out_ref[...] = o_ref[0] + o_ref[1] + b_ref[...]

    return pl.pallas_call(
        body,
        grid=(_N // _BM,),
        in_specs=[pl.BlockSpec((2, _BM, 64), lambda i: (0, i, 0)),
                  pl.BlockSpec((1, 64), lambda i: (0, 0))],
        out_specs=pl.BlockSpec((_BM, 64), lambda i: (i, 0)),
        out_shape=jax.ShapeDtypeStruct((_N, 64), jnp.float32),
    )(op, b2r)


def kernel(x, adj_row, adj_col, adj_val, W1, b1, W2, b2):
    y = _mm1(x, W1)
    zp = _spmm128(y, adj_row, adj_col, adj_val)
    u = _mm2(zp, W2, b1.reshape(1, -1))
    op = _spmm64(u, adj_row, adj_col, adj_val)
    return _add2(op, b2.reshape(1, -1))


# speedup vs baseline: 8.5303x; 2.1572x over previous
import jax
import jax.numpy as jnp
from jax import lax
from jax.experimental import pallas as pl
from jax.experimental.pallas import tpu as pltpu
from jax.experimental.pallas import tpu_sc as plsc

_N = 10000
_E = 320000
_NC = 2
_NS = 16
_NW = _NC * _NS
_EPW = _E // _NW
_C = 80
_NCHUNK = _EPW // _C
_RPT = _N // _NS
_ZR = 125
_L = 16


def _make_spmm(feat):
    grp = feat // _L
    mesh = plsc.VectorSubcoreMesh(core_axis_name="c", subcore_axis_name="s")

    def body(y_hbm, row_hbm, col_hbm, val_hbm, out_hbm,
             col2d, val2d, row2d, rows0, rows1, acc,
             msem, g0, g1, s0, s1):
        cid = lax.axis_index("c")
        sid = lax.axis_index("s")
        wid = cid * _NS + sid

        pltpu.async_copy(col_hbm.at[wid], col2d, msem)
        pltpu.async_copy(val_hbm.at[wid], val2d, msem)
        pltpu.async_copy(row_hbm.at[wid], row2d, msem)

        @pl.loop(0, _C)
        def _(r):
            for k in range(grp):
                rows0[r, pl.ds(k * _L, _L)] = jnp.zeros((_L,), jnp.float32)

        for z in range(_RPT // _C):
            pltpu.sync_copy(rows0, acc.at[pl.ds(sid * _RPT + z * _C, _C)])
        _zr = _RPT % _C
        pltpu.sync_copy(rows0.at[pl.ds(0, _zr)],
                        acc.at[pl.ds(sid * _RPT + _RPT - _zr, _zr)])

        pltpu.make_async_copy(col_hbm.at[wid], col2d, msem).wait()
        pltpu.make_async_copy(val_hbm.at[wid], val2d, msem).wait()
        pltpu.make_async_copy(row_hbm.at[wid], row2d, msem).wait()
        plsc.subcore_barrier()

        bufs = (rows0, rows1)
        gsems = (g0, g1)
        ssems = (s0, s1)

        def g_start(ch, p):
            pltpu.async_copy(y_hbm.at[col2d.at[ch]], bufs[p], gsems[p])

        def g_wait(ch, p):
            pltpu.make_async_copy(y_hbm.at[col2d.at[ch]], bufs[p],
                                  gsems[p]).wait()

        def s_start(ch, p):
            pltpu.async_copy(bufs[p], acc.at[row2d.at[ch]], ssems[p],
                             add=True)

        def s_wait(ch, p):
            pltpu.make_async_copy(bufs[p], acc.at[row2d.at[ch]],
                                  ssems[p]).wait()

        def scale(ch, p):
            buf = bufs[p]

            @pl.loop(0, _C // _L)
            def _(j0):
                vals16 = val2d[ch, pl.ds(j0 * _L, _L)]
                for t in range(_L):
                    v = jnp.take(vals16, jnp.full((_L,), t, jnp.int32))
                    j = j0 * _L + t
                    for k in range(grp):
                        sl = pl.ds(k * _L, _L)
                        buf[j, sl] = buf[j, sl] * v

        g_start(0, 0)
        g_start(1, 1)

        @pl.loop(0, _NCHUNK - 1, step=2)
        def _(c):
            g_wait(c, 0)
            scale(c, 0)
            s_start(c, 0)
            g_wait(c + 1, 1)
            scale(c + 1, 1)
            s_wait(c, 0)
            g_start(c + 2, 0)
            s_start(c + 1, 1)

            @pl.when(c + 3 < _NCHUNK)
            def _():
                s_wait(c + 1, 1)
                g_start(c + 3, 1)

        last = _NCHUNK - 1
        g_wait(last, 0)
        scale(last, 0)
        s_start(last, 0)
        s_wait(last - 1, 1)
        s_wait(last, 0)

        plsc.subcore_barrier()
        pltpu.sync_copy(acc.at[pl.ds(sid * _RPT, _RPT)],
                        out_hbm.at[cid, pl.ds(sid * _RPT, _RPT)])

    return pl.kernel(
        body,
        out_type=jax.ShapeDtypeStruct((_NC, _N, feat), jnp.float32),
        mesh=mesh,
        scratch_types=[
            pltpu.VMEM((_NCHUNK, _C), jnp.int32),
            pltpu.VMEM((_NCHUNK, _C), jnp.float32),
            pltpu.VMEM((_NCHUNK, _C), jnp.int32),
            pltpu.VMEM((_C, feat), jnp.float32),
            pltpu.VMEM((_C, feat), jnp.float32),
            pltpu.VMEM_SHARED((_N, feat), jnp.float32),
            pltpu.SemaphoreType.DMA,
            pltpu.SemaphoreType.DMA,
            pltpu.SemaphoreType.DMA,
            pltpu.SemaphoreType.DMA,
            pltpu.SemaphoreType.DMA,
        ],
        compiler_params=pltpu.CompilerParams(use_tc_tiling_on_sc=False),
    )


_spmm128 = _make_spmm(128)
_spmm64 = _make_spmm(64)

_BM = 2000


def _mm1(x, w1):
    def body(x_ref, w_ref, o_ref):
        o_ref[...] = lax.dot_general(
            x_ref[...], w_ref[...], (((1,), (1,)), ((), ())),
            preferred_element_type=jnp.float32)

    return pl.pallas_call(
        body,
        grid=(_N // _BM,),
        in_specs=[pl.BlockSpec((_BM, 128), lambda i: (i, 0)),
                  pl.BlockSpec((128, 128), lambda i: (0, 0))],
        out_specs=pl.BlockSpec((_BM, 128), lambda i: (i, 0)),
        out_shape=jax.ShapeDtypeStruct((_N, 128), jnp.float32),
    )(x, w1)


def _mm2(zp, w2, b1r):
    def body(z_ref, w_ref, b_ref, o_ref):
        z = z_ref[0] + z_ref[1] + b_ref[...]
        o_ref[...] = lax.dot_general(
            z, w_ref[...], (((1,), (1,)), ((), ())),
            preferred_element_type=jnp.float32)

    return pl.pallas_call(
        body,
        grid=(_N // _BM,),
        in_specs=[pl.BlockSpec((2, _BM, 128), lambda i: (0, i, 0)),
                  pl.BlockSpec((64, 128), lambda i: (0, 0)),
                  pl.BlockSpec((1, 128), lambda i: (0, 0))],
        out_specs=pl.BlockSpec((_BM, 64), lambda i: (i, 0)),
        out_shape=jax.ShapeDtypeStruct((_N, 64), jnp.float32),
    )(zp, w2, b1r)


def _add2(op, b2r):
    def body(o_ref, b_ref, out_ref):
        out_ref[...] = o_ref[0] + o_ref[1] + b_ref[...]

    return pl.pallas_call(
        body,
        grid=(_N // _BM,),
        in_specs=[pl.BlockSpec((2, _BM, 64), lambda i: (0, i, 0)),
                  pl.BlockSpec((1, 64), lambda i: (0, 0))],
        out_specs=pl.BlockSpec((_BM, 64), lambda i: (i, 0)),
        out_shape=jax.ShapeDtypeStruct((_N, 64), jnp.float32),
    )(op, b2r)


def kernel(x, adj_row, adj_col, adj_val, W1, b1, W2, b2):
    row3 = adj_row.reshape(_NW, _NCHUNK, _C)
    col3 = adj_col.reshape(_NW, _NCHUNK, _C)
    val3 = adj_val.reshape(_NW, _NCHUNK, _C)
    y = _mm1(x, W1)
    zp = _spmm128(y, row3, col3, val3)
    u = _mm2(zp, W2, b1.reshape(1, -1))
    op = _spmm64(u, row3, col3, val3)
    return _add2(op, b2.reshape(1, -1))
